# baseline (speedup 1.0000x reference)
import jax
import jax.numpy as jnp
from jax.experimental import pallas as pl
from jax.experimental.pallas import tpu as pltpu

NUM_NODES = 1000
FEAT = 64
HID = 64
OUT = 32
BATCH = 16
NTOT = BATCH * NUM_NODES
NCHUNK = 8
CHUNK = NTOT // NCHUNK
BPC = BATCH // NCHUNK


def _gcn_kernel(x_hbm, w1_ref, b1_ref, w2_ref, b2_ref, w3_ref, b3_ref,
                wl_ref, bl_ref, out_ref, xv_ref, sem):
    def _copy(i):
        return pltpu.make_async_copy(
            x_hbm.at[pl.ds(i * CHUNK, CHUNK), :],
            xv_ref.at[pl.ds(i * CHUNK, CHUNK), :],
            sem.at[i],
        )

    for i in range(NCHUNK):
        _copy(i).start()

    pooled_parts = []
    for i in range(NCHUNK):
        _copy(i).wait()
        h = xv_ref[i * CHUNK:(i + 1) * CHUNK]
        if i == 0:
            m0 = jnp.mean(h[0:NUM_NODES], axis=0, keepdims=True)
            row = jax.lax.broadcasted_iota(jnp.int32, (CHUNK, 1), 0)
            h = jnp.where(row < NUM_NODES, m0, h)
        h = jnp.dot(h, w1_ref[...], preferred_element_type=jnp.float32) + b1_ref[...]
        h = jnp.maximum(h, 0.0)
        h = jnp.dot(h, w2_ref[...], preferred_element_type=jnp.float32) + b2_ref[...]
        h = jnp.maximum(h, 0.0)
        bidx = jax.lax.broadcasted_iota(jnp.int32, (BPC, CHUNK), 0)
        nidx = jax.lax.broadcasted_iota(jnp.int32, (BPC, CHUNK), 1)
        pool = jnp.where(nidx // NUM_NODES == bidx, 1.0 / NUM_NODES, 0.0)
        pooled_parts.append(jnp.dot(pool, h, preferred_element_type=jnp.float32))

    pooled = jnp.concatenate(pooled_parts, axis=0)
    o = jnp.dot(pooled, w3_ref[...], preferred_element_type=jnp.float32) + b3_ref[...]
    out_ref[...] = (
        jnp.dot(o, wl_ref[...], preferred_element_type=jnp.float32) + bl_ref[...]
    )


@jax.jit
def _run(x, W1, b1, W2, b2, W3, b3, Wl, bl):
    B = x.shape[0]
    x = x.astype(jnp.float32).reshape(B * NUM_NODES, FEAT)
    b1 = b1.reshape(1, HID)
    b2 = b2.reshape(1, HID)
    b3 = b3.reshape(1, HID)
    bl = bl.reshape(1, OUT)
    vmem = pl.BlockSpec(memory_space=pltpu.MemorySpace.VMEM)
    return pl.pallas_call(
        _gcn_kernel,
        in_specs=[pl.BlockSpec(memory_space=pl.ANY),
                  vmem, vmem, vmem, vmem, vmem, vmem, vmem, vmem],
        out_specs=pl.BlockSpec(memory_space=pltpu.MemorySpace.VMEM),
        out_shape=jax.ShapeDtypeStruct((B, OUT), jnp.float32),
        scratch_shapes=[pltpu.VMEM((NTOT, FEAT), jnp.float32),
                        pltpu.SemaphoreType.DMA((NCHUNK,))],
    )(x, W1, b1, W2, b2, W3, b3, Wl, bl)


def kernel(x, W1, b1, W2, b2, W3, b3, Wl, bl, batch_size=BATCH, device=0):
    return _run(x, W1, b1, W2, b2, W3, b3, Wl, bl)


# speedup vs baseline: 1.0784x; 1.0784x over previous
import jax
import jax.numpy as jnp
from jax.experimental import pallas as pl
from jax.experimental.pallas import tpu as pltpu

NUM_NODES = 1000
FEAT = 64
HID = 64
OUT = 32
BATCH = 16
NTOT = BATCH * NUM_NODES
NCHUNK = 4
CHUNK = NTOT // NCHUNK
BPC = BATCH // NCHUNK


def _gcn_kernel(x_hbm, w1_ref, b1_ref, w2_ref, b2_ref, w3_ref, b3_ref,
                wl_ref, bl_ref, out_ref, *scratch):
    xv = scratch[:NCHUNK]
    sem = scratch[NCHUNK]

    def _copy(i):
        return pltpu.make_async_copy(
            x_hbm.at[pl.ds(i * CHUNK, CHUNK), :], xv[i], sem.at[i])

    for i in range(NCHUNK):
        _copy(i).start()

    pooled_parts = []
    for i in range(NCHUNK):
        _copy(i).wait()
        h = xv[i][...]
        if i == 0:
            m0 = jnp.mean(h[0:NUM_NODES], axis=0, keepdims=True)
            row = jax.lax.broadcasted_iota(jnp.int32, (CHUNK, 1), 0)
            h = jnp.where(row < NUM_NODES, m0, h)
        h = jnp.dot(h, w1_ref[...], preferred_element_type=jnp.float32) + b1_ref[...]
        h = jnp.maximum(h, 0.0)
        h = jnp.dot(h, w2_ref[...], preferred_element_type=jnp.float32) + b2_ref[...]
        h = jnp.maximum(h, 0.0)
        bidx = jax.lax.broadcasted_iota(jnp.int32, (BPC, CHUNK), 0)
        nidx = jax.lax.broadcasted_iota(jnp.int32, (BPC, CHUNK), 1)
        pool = jnp.where(nidx // NUM_NODES == bidx, 1.0 / NUM_NODES, 0.0)
        pooled_parts.append(jnp.dot(pool, h, preferred_element_type=jnp.float32))

    pooled = jnp.concatenate(pooled_parts, axis=0)
    o = jnp.dot(pooled, w3_ref[...], preferred_element_type=jnp.float32) + b3_ref[...]
    out_ref[...] = (
        jnp.dot(o, wl_ref[...], preferred_element_type=jnp.float32) + bl_ref[...]
    )


@jax.jit
def _run(x, W1, b1, W2, b2, W3, b3, Wl, bl):
    B = x.shape[0]
    x = x.astype(jnp.float32).reshape(B * NUM_NODES, FEAT)
    b1 = b1.reshape(1, HID)
    b2 = b2.reshape(1, HID)
    b3 = b3.reshape(1, HID)
    bl = bl.reshape(1, OUT)
    vmem = pl.BlockSpec(memory_space=pltpu.MemorySpace.VMEM)
    return pl.pallas_call(
        _gcn_kernel,
        in_specs=[pl.BlockSpec(memory_space=pl.ANY),
                  vmem, vmem, vmem, vmem, vmem, vmem, vmem, vmem],
        out_specs=pl.BlockSpec(memory_space=pltpu.MemorySpace.VMEM),
        out_shape=jax.ShapeDtypeStruct((B, OUT), jnp.float32),
        scratch_shapes=[pltpu.VMEM((CHUNK, FEAT), jnp.float32)] * NCHUNK
        + [pltpu.SemaphoreType.DMA((NCHUNK,))],
    )(x, W1, b1, W2, b2, W3, b3, Wl, bl)


def kernel(x, W1, b1, W2, b2, W3, b3, Wl, bl, batch_size=BATCH, device=0):
    return _run(x, W1, b1, W2, b2, W3, b3, Wl, bl)
